# baseline (speedup 1.0000x reference)
import dataclasses
import functools

import jax
import jax.numpy as jnp
from jax import lax
from jax.experimental import pallas as pl
from jax.experimental.pallas import tpu as pltpu
from jax.experimental.pallas import tpu_sc as plsc

D = 128
CH = 128
NC = 2
NS = 16
L = 16
R = 1024


def _mesh():
    return plsc.VectorSubcoreMesh(core_axis_name="c", subcore_axis_name="s")


def _no_layout_params():
    cp = pltpu.CompilerParams()
    if "needs_layout_passes" in pltpu.CompilerParams.__dataclass_fields__:
        cp = dataclasses.replace(cp, needs_layout_passes=False)
    return cp


def _make_deg_kernel(e_pad, n_pad):
    edges_per_tile = e_pad // (NC * NS)
    chunks = edges_per_tile // CH
    hrows = n_pad // D
    wsub = hrows // 8

    @functools.partial(
        pl.kernel,
        out_type=jax.ShapeDtypeStruct((NC * hrows, D), jnp.float32),
        mesh=_mesh(),
        compiler_params=_no_layout_params(),
        scratch_types=[
            pltpu.VMEM((CH,), jnp.int32),
            pltpu.VMEM((hrows,), jnp.int32),
            pltpu.VMEM((hrows, D), jnp.float32),
            pltpu.VMEM_SHARED((hrows, D), jnp.float32),
            pltpu.SemaphoreType.DMA,
        ],
    )
    def deg_kernel(dst_hbm, lin_hbm, zeros_hbm, out_hbm,
                   didx, lin_v, hist, acc, sem):
        c = lax.axis_index("c")
        s = lax.axis_index("s")

        @pl.when(s == 0)
        def _():
            pltpu.sync_copy(zeros_hbm, acc)

        pltpu.sync_copy(lin_hbm, lin_v)
        pltpu.sync_copy(zeros_hbm, hist)
        plsc.subcore_barrier()

        base = (c * NS + s) * edges_per_tile
        ones16 = jnp.full((L,), 1.0, jnp.float32)

        @pl.loop(0, chunks)
        def _(j):
            pltpu.sync_copy(dst_hbm.at[pl.ds(base + j * CH, CH)], didx)
            for k in range(CH // L):
                v = didx[pl.ds(k * L, L)]
                row = lax.shift_right_logical(v, 7)
                col = lax.bitwise_and(v, 127)
                plsc.addupdate_scatter(hist, [row, col], ones16)

        pltpu.sync_copy(hist, acc.at[lin_v], add=True)
        plsc.subcore_barrier()

        @pl.when(s < wsub)
        def _():
            pltpu.sync_copy(
                acc.at[pl.ds(s * 8, 8)],
                out_hbm.at[pl.ds(c * hrows + s * 8, 8)],
            )

    return deg_kernel


def _make_agg_kernel(e_pad, n_pad):
    edges_per_tile = e_pad // (NC * NS)
    chunks = edges_per_tile // CH
    pairs = chunks // 2
    rows_per_sub = n_pad // NS

    @functools.partial(
        pl.kernel,
        out_type=jax.ShapeDtypeStruct((NC * n_pad, D), jnp.float32),
        mesh=_mesh(),
        scratch_types=[
            pltpu.VMEM((CH,), jnp.int32),
            pltpu.VMEM((CH,), jnp.int32),
            pltpu.VMEM((CH, D), jnp.float32),
            pltpu.VMEM_SHARED((n_pad, D), jnp.float32),
            pltpu.SemaphoreType.DMA,
        ],
    )
    def agg_kernel(src_hbm, dst_hbm, hs_hbm, zeros_hbm, out_hbm,
                   sidx, didx, rows, acc, sem):
        c = lax.axis_index("c")
        s = lax.axis_index("s")

        @pl.when(s == 0)
        def _():
            pltpu.sync_copy(zeros_hbm, acc)

        plsc.subcore_barrier()

        base = (c * NS + s) * edges_per_tile

        @pl.loop(0, chunks)
        def _(j):
            off = base + j * CH
            pltpu.sync_copy(src_hbm.at[pl.ds(off, CH)], sidx)
            pltpu.sync_copy(dst_hbm.at[pl.ds(off, CH)], didx)
            pltpu.async_copy(hs_hbm.at[sidx], rows, sem).wait()
            pltpu.sync_copy(rows, acc.at[didx], add=True)

        plsc.subcore_barrier()
        pltpu.sync_copy(
            acc.at[pl.ds(s * rows_per_sub, rows_per_sub)],
            out_hbm.at[pl.ds(c * n_pad + s * rows_per_sub, rows_per_sub)],
        )

    return agg_kernel


def _dinv_block(deg_ref):
    d = deg_ref[0] + deg_ref[1] + 1.0
    dinv = lax.rsqrt(d)
    eye = (lax.broadcasted_iota(jnp.int32, (D, D), 0)
           == lax.broadcasted_iota(jnp.int32, (D, D), 1)).astype(jnp.float32)
    dt = lax.dot_general(eye, dinv, (((1,), (1,)), ((), ())),
                         precision=lax.Precision.HIGHEST,
                         preferred_element_type=jnp.float32)
    blks = [jnp.broadcast_to(dt[:, k:k + 1], (D, D))
            for k in range(R // D)]
    return jnp.concatenate(blks, axis=0)


def _matmul_body(x_ref, w_ref, o_ref):
    o_ref[...] = jnp.dot(x_ref[...], w_ref[...],
                         preferred_element_type=jnp.float32)


def _scale_body(deg_ref, h_ref, o_ref):
    o_ref[...] = h_ref[...] * _dinv_block(deg_ref)


def _mid_body(deg_ref, agg_ref, hs_ref, w_ref, b_ref, o_ref):
    dinv = _dinv_block(deg_ref)
    t = (agg_ref[0] + agg_ref[1] + hs_ref[...]) * dinv + b_ref[...]
    o_ref[...] = jnp.dot(t, w_ref[...],
                         preferred_element_type=jnp.float32) * dinv


def _final_body(deg_ref, agg_ref, hs_ref, b_ref, o_ref):
    o_ref[...] = (agg_ref[0] + agg_ref[1] + hs_ref[...]) * _dinv_block(deg_ref) \
        + b_ref[...]


def kernel(x, edge_index, W1, b1, W2, b2):
    N = x.shape[0]
    E = edge_index.shape[1]
    n_pad = ((N + R - 1) // R) * R
    group = NC * NS * CH * 2
    e_pad = ((E + group - 1) // group) * group
    pad = e_pad - E
    hrows = n_pad // D

    src = jnp.concatenate([edge_index[0], jnp.zeros((pad,), jnp.int32)])
    dst = jnp.concatenate([edge_index[1], jnp.full((pad,), N, jnp.int32)])
    x_pad = jnp.pad(x, ((0, n_pad - N), (0, 0)))
    zeros_agg = jnp.zeros((n_pad, D), jnp.float32)
    zeros_deg = jnp.zeros((hrows, D), jnp.float32)
    lin = jnp.arange(hrows, dtype=jnp.int32)
    w1t = W1.T
    w2t = W2.T
    b1r = b1.reshape(1, D)
    b2r = b2.reshape(1, D)

    deg_kernel = _make_deg_kernel(e_pad, n_pad)
    agg_kernel = _make_agg_kernel(e_pad, n_pad)
    grid = (n_pad // R,)

    deg2 = deg_kernel(dst, lin, zeros_deg).reshape(NC, hrows, D)

    h1 = pl.pallas_call(
        _matmul_body,
        grid=grid,
        in_specs=[pl.BlockSpec((R, D), lambda i: (i, 0)),
                  pl.BlockSpec((D, D), lambda i: (0, 0))],
        out_specs=pl.BlockSpec((R, D), lambda i: (i, 0)),
        out_shape=jax.ShapeDtypeStruct((n_pad, D), jnp.float32),
    )(x_pad, w1t)

    row_spec = pl.BlockSpec((R, D), lambda i: (i, 0))
    agg_spec = pl.BlockSpec((NC, R, D), lambda i: (0, i, 0))
    deg_spec = pl.BlockSpec((NC, R // D, D), lambda i: (0, i, 0))
    b_spec = pl.BlockSpec((1, D), lambda i: (0, 0))
    w_spec = pl.BlockSpec((D, D), lambda i: (0, 0))

    hs1 = pl.pallas_call(
        _scale_body,
        grid=grid,
        in_specs=[deg_spec, row_spec],
        out_specs=row_spec,
        out_shape=jax.ShapeDtypeStruct((n_pad, D), jnp.float32),
    )(deg2, h1)

    agg1 = agg_kernel(src, dst, hs1, zeros_agg).reshape(NC, n_pad, D)

    hs2 = pl.pallas_call(
        _mid_body,
        grid=grid,
        in_specs=[deg_spec, agg_spec, row_spec, w_spec, b_spec],
        out_specs=row_spec,
        out_shape=jax.ShapeDtypeStruct((n_pad, D), jnp.float32),
    )(deg2, agg1, hs1, w2t, b1r)

    agg2 = agg_kernel(src, dst, hs2, zeros_agg).reshape(NC, n_pad, D)

    out = pl.pallas_call(
        _final_body,
        grid=grid,
        in_specs=[deg_spec, agg_spec, row_spec, b_spec],
        out_specs=row_spec,
        out_shape=jax.ShapeDtypeStruct((n_pad, D), jnp.float32),
    )(deg2, agg2, hs2, b2r)

    return out[:N]


# speedup vs baseline: 1.5806x; 1.5806x over previous
import dataclasses
import functools

import jax
import jax.numpy as jnp
from jax import lax
from jax.experimental import pallas as pl
from jax.experimental.pallas import tpu as pltpu
from jax.experimental.pallas import tpu_sc as plsc

D = 128
CH = 128
NC = 2
NS = 16
L = 16
R = 1024


def _mesh():
    return plsc.VectorSubcoreMesh(core_axis_name="c", subcore_axis_name="s")


def _no_layout_params():
    cp = pltpu.CompilerParams()
    if "needs_layout_passes" in pltpu.CompilerParams.__dataclass_fields__:
        cp = dataclasses.replace(cp, needs_layout_passes=False)
    return cp


def _make_deg_kernel(e_pad, n_pad):
    edges_per_tile = e_pad // (NC * NS)
    chunks = edges_per_tile // CH
    hrows = n_pad // D
    wsub = hrows // 8

    @functools.partial(
        pl.kernel,
        out_type=jax.ShapeDtypeStruct((NC * hrows, D), jnp.float32),
        mesh=_mesh(),
        compiler_params=_no_layout_params(),
        scratch_types=[
            pltpu.VMEM((CH,), jnp.int32),
            pltpu.VMEM((hrows,), jnp.int32),
            pltpu.VMEM((hrows, D), jnp.float32),
            pltpu.VMEM_SHARED((hrows, D), jnp.float32),
            pltpu.SemaphoreType.DMA,
        ],
    )
    def deg_kernel(dst_hbm, lin_hbm, zeros_hbm, out_hbm,
                   didx, lin_v, hist, acc, sem):
        c = lax.axis_index("c")
        s = lax.axis_index("s")

        @pl.when(s == 0)
        def _():
            pltpu.sync_copy(zeros_hbm, acc)

        pltpu.sync_copy(lin_hbm, lin_v)
        pltpu.sync_copy(zeros_hbm, hist)
        plsc.subcore_barrier()

        base = (c * NS + s) * edges_per_tile
        ones16 = jnp.full((L,), 1.0, jnp.float32)

        @pl.loop(0, chunks)
        def _(j):
            pltpu.sync_copy(dst_hbm.at[pl.ds(base + j * CH, CH)], didx)
            for k in range(CH // L):
                v = didx[pl.ds(k * L, L)]
                row = lax.shift_right_logical(v, 7)
                col = lax.bitwise_and(v, 127)
                plsc.addupdate_scatter(hist, [row, col], ones16)

        pltpu.sync_copy(hist, acc.at[lin_v], add=True)
        plsc.subcore_barrier()

        @pl.when(s < wsub)
        def _():
            pltpu.sync_copy(
                acc.at[pl.ds(s * 8, 8)],
                out_hbm.at[pl.ds(c * hrows + s * 8, 8)],
            )

    return deg_kernel


def _make_agg_kernel(e_pad, n_pad):
    edges_per_tile = e_pad // (NC * NS)
    chunks = edges_per_tile // CH
    pairs = chunks // 2
    rows_per_sub = n_pad // NS

    @functools.partial(
        pl.kernel,
        out_type=jax.ShapeDtypeStruct((NC * n_pad, D), jnp.float32),
        mesh=_mesh(),
        scratch_types=[
            pltpu.VMEM((CH,), jnp.int32),
            pltpu.VMEM((CH,), jnp.int32),
            pltpu.VMEM((CH, D), jnp.float32),
            pltpu.VMEM_SHARED((n_pad, D), jnp.float32),
            pltpu.SemaphoreType.DMA,
        ],
    )
    def agg_kernel(src_hbm, dst_hbm, hs_hbm, zeros_hbm, out_hbm,
                   sidx, didx, rows, acc, sem):
        c = lax.axis_index("c")
        s = lax.axis_index("s")

        @pl.when(s == 0)
        def _():
            pltpu.sync_copy(zeros_hbm, acc)

        plsc.subcore_barrier()

        base = (c * NS + s) * edges_per_tile

        @pl.loop(0, chunks)
        def _(j):
            off = base + j * CH
            pltpu.sync_copy(src_hbm.at[pl.ds(off, CH)], sidx)
            pltpu.sync_copy(dst_hbm.at[pl.ds(off, CH)], didx)
            pltpu.async_copy(hs_hbm.at[sidx], rows, sem).wait()
            pltpu.sync_copy(rows, acc.at[didx], add=True)

        plsc.subcore_barrier()
        pltpu.sync_copy(
            acc.at[pl.ds(s * rows_per_sub, rows_per_sub)],
            out_hbm.at[pl.ds(c * n_pad + s * rows_per_sub, rows_per_sub)],
        )

    return agg_kernel


def _dinv_block(deg_ref):
    d = deg_ref[0] + deg_ref[1] + 1.0
    dinv = lax.rsqrt(d)
    eye = (lax.broadcasted_iota(jnp.int32, (D, D), 0)
           == lax.broadcasted_iota(jnp.int32, (D, D), 1)).astype(jnp.float32)
    dt = lax.dot_general(eye, dinv, (((1,), (1,)), ((), ())),
                         precision=lax.Precision.HIGHEST,
                         preferred_element_type=jnp.float32)
    blks = [jnp.broadcast_to(dt[:, k:k + 1], (D, D))
            for k in range(R // D)]
    return jnp.concatenate(blks, axis=0)


def _matmul_body(x_ref, w_ref, o_ref):
    o_ref[...] = jnp.dot(x_ref[...], w_ref[...],
                         preferred_element_type=jnp.float32)


def _scale_body(deg_ref, h_ref, o_ref):
    o_ref[...] = h_ref[...] * _dinv_block(deg_ref)


def _mid_body(deg_ref, agg_ref, hs_ref, w_ref, b_ref, o_ref):
    dinv = _dinv_block(deg_ref)
    t = (agg_ref[0] + agg_ref[1] + hs_ref[...]) * dinv + b_ref[...]
    o_ref[...] = jnp.dot(t, w_ref[...],
                         preferred_element_type=jnp.float32) * dinv


def _final_body(deg_ref, agg_ref, hs_ref, b_ref, o_ref):
    o_ref[...] = (agg_ref[0] + agg_ref[1] + hs_ref[...]) * _dinv_block(deg_ref) \
        + b_ref[...]


def kernel(x, edge_index, W1, b1, W2, b2):
    N = x.shape[0]
    E = edge_index.shape[1]
    n_pad = ((N + R - 1) // R) * R
    group = NC * NS * CH
    e_pad = ((E + group - 1) // group) * group
    pad = e_pad - E
    hrows = n_pad // D

    src = jnp.concatenate([edge_index[0], jnp.zeros((pad,), jnp.int32)])
    dst = jnp.concatenate([edge_index[1], jnp.full((pad,), N, jnp.int32)])
    x_pad = jnp.pad(x, ((0, n_pad - N), (0, 0)))
    zeros_agg = jnp.zeros((n_pad, D), jnp.float32)
    zeros_deg = jnp.zeros((hrows, D), jnp.float32)
    lin = jnp.arange(hrows, dtype=jnp.int32)
    w1t = W1.T
    w2t = W2.T
    b1r = b1.reshape(1, D)
    b2r = b2.reshape(1, D)

    deg_kernel = _make_deg_kernel(e_pad, n_pad)
    agg_kernel = _make_agg_kernel(e_pad, n_pad)
    grid = (n_pad // R,)

    deg2 = deg_kernel(dst, lin, zeros_deg).reshape(NC, hrows, D)

    h1 = pl.pallas_call(
        _matmul_body,
        grid=grid,
        in_specs=[pl.BlockSpec((R, D), lambda i: (i, 0)),
                  pl.BlockSpec((D, D), lambda i: (0, 0))],
        out_specs=pl.BlockSpec((R, D), lambda i: (i, 0)),
        out_shape=jax.ShapeDtypeStruct((n_pad, D), jnp.float32),
    )(x_pad, w1t)

    row_spec = pl.BlockSpec((R, D), lambda i: (i, 0))
    agg_spec = pl.BlockSpec((NC, R, D), lambda i: (0, i, 0))
    deg_spec = pl.BlockSpec((NC, R // D, D), lambda i: (0, i, 0))
    b_spec = pl.BlockSpec((1, D), lambda i: (0, 0))
    w_spec = pl.BlockSpec((D, D), lambda i: (0, 0))

    hs1 = pl.pallas_call(
        _scale_body,
        grid=grid,
        in_specs=[deg_spec, row_spec],
        out_specs=row_spec,
        out_shape=jax.ShapeDtypeStruct((n_pad, D), jnp.float32),
    )(deg2, h1)

    agg1 = agg_kernel(src, dst, hs1, zeros_agg).reshape(NC, n_pad, D)

    hs2 = pl.pallas_call(
        _mid_body,
        grid=grid,
        in_specs=[deg_spec, agg_spec, row_spec, w_spec, b_spec],
        out_specs=row_spec,
        out_shape=jax.ShapeDtypeStruct((n_pad, D), jnp.float32),
    )(deg2, agg1, hs1, w2t, b1r)

    agg2 = agg_kernel(src, dst, hs2, zeros_agg).reshape(NC, n_pad, D)

    out = pl.pallas_call(
        _final_body,
        grid=grid,
        in_specs=[deg_spec, agg_spec, row_spec, b_spec],
        out_specs=row_spec,
        out_shape=jax.ShapeDtypeStruct((n_pad, D), jnp.float32),
    )(deg2, agg2, hs2, b2r)

    return out[:N]


# speedup vs baseline: 1.5809x; 1.0002x over previous
import dataclasses
import functools

import jax
import jax.numpy as jnp
from jax import lax
from jax.experimental import pallas as pl
from jax.experimental.pallas import tpu as pltpu
from jax.experimental.pallas import tpu_sc as plsc

D = 128
CH = 128
NC = 2
NS = 16
L = 16
R = 1024


def _mesh():
    return plsc.VectorSubcoreMesh(core_axis_name="c", subcore_axis_name="s")


def _no_layout_params():
    cp = pltpu.CompilerParams()
    if "needs_layout_passes" in pltpu.CompilerParams.__dataclass_fields__:
        cp = dataclasses.replace(cp, needs_layout_passes=False)
    return cp


def _make_deg_kernel(e_pad, n_pad):
    edges_per_tile = e_pad // (NC * NS)
    chunks = edges_per_tile // CH
    hrows = n_pad // D
    wsub = hrows // 8

    @functools.partial(
        pl.kernel,
        out_type=jax.ShapeDtypeStruct((NC * hrows, D), jnp.float32),
        mesh=_mesh(),
        compiler_params=_no_layout_params(),
        scratch_types=[
            pltpu.VMEM((CH,), jnp.int32),
            pltpu.VMEM((hrows,), jnp.int32),
            pltpu.VMEM((hrows, D), jnp.float32),
            pltpu.VMEM_SHARED((hrows, D), jnp.float32),
            pltpu.SemaphoreType.DMA,
        ],
    )
    def deg_kernel(dst_hbm, lin_hbm, zeros_hbm, out_hbm,
                   didx, lin_v, hist, acc, sem):
        c = lax.axis_index("c")
        s = lax.axis_index("s")

        @pl.when(s == 0)
        def _():
            pltpu.sync_copy(zeros_hbm, acc)

        pltpu.sync_copy(lin_hbm, lin_v)
        pltpu.sync_copy(zeros_hbm, hist)
        plsc.subcore_barrier()

        base = (c * NS + s) * edges_per_tile
        ones16 = jnp.full((L,), 1.0, jnp.float32)

        @pl.loop(0, chunks)
        def _(j):
            pltpu.sync_copy(dst_hbm.at[pl.ds(base + j * CH, CH)], didx)
            for k in range(CH // L):
                v = didx[pl.ds(k * L, L)]
                row = lax.shift_right_logical(v, 7)
                col = lax.bitwise_and(v, 127)
                plsc.addupdate_scatter(hist, [row, col], ones16)

        pltpu.sync_copy(hist, acc.at[lin_v], add=True)
        plsc.subcore_barrier()

        @pl.when(s < wsub)
        def _():
            pltpu.sync_copy(
                acc.at[pl.ds(s * 8, 8)],
                out_hbm.at[pl.ds(c * hrows + s * 8, 8)],
            )

    return deg_kernel


def _make_agg_kernel(e_pad, n_pad):
    edges_per_tile = e_pad // (NC * NS)
    chunks = edges_per_tile // CH
    pairs = chunks // 2
    rows_per_sub = n_pad // NS

    @functools.partial(
        pl.kernel,
        out_type=jax.ShapeDtypeStruct((NC * n_pad, D), jnp.float32),
        mesh=_mesh(),
        scratch_types=[
            pltpu.VMEM((CH,), jnp.int32),
            pltpu.VMEM((CH,), jnp.int32),
            pltpu.VMEM((CH, D), jnp.float32),
            pltpu.VMEM_SHARED((n_pad, D), jnp.float32),
            pltpu.SemaphoreType.DMA,
        ],
    )
    def agg_kernel(src_hbm, dst_hbm, hs0_hbm, hs1_hbm, zeros_hbm, out_hbm,
                   sidx, didx, rows, acc, sem):
        c = lax.axis_index("c")
        s = lax.axis_index("s")

        @pl.when(s == 0)
        def _():
            pltpu.sync_copy(zeros_hbm, acc)

        plsc.subcore_barrier()

        base = (c * NS + s) * edges_per_tile

        def run(hs_hbm):
            @pl.loop(0, chunks)
            def _(j):
                off = base + j * CH
                pltpu.sync_copy(src_hbm.at[pl.ds(off, CH)], sidx)
                pltpu.sync_copy(dst_hbm.at[pl.ds(off, CH)], didx)
                pltpu.async_copy(hs_hbm.at[sidx], rows, sem).wait()
                pltpu.sync_copy(rows, acc.at[didx], add=True)

        @pl.when(c == 0)
        def _():
            run(hs0_hbm)

        @pl.when(c == 1)
        def _():
            run(hs1_hbm)

        plsc.subcore_barrier()
        pltpu.sync_copy(
            acc.at[pl.ds(s * rows_per_sub, rows_per_sub)],
            out_hbm.at[pl.ds(c * n_pad + s * rows_per_sub, rows_per_sub)],
        )

    return agg_kernel


def _dinv_block(deg_ref):
    d = deg_ref[0] + deg_ref[1] + 1.0
    dinv = lax.rsqrt(d)
    eye = (lax.broadcasted_iota(jnp.int32, (D, D), 0)
           == lax.broadcasted_iota(jnp.int32, (D, D), 1)).astype(jnp.float32)
    dt = lax.dot_general(eye, dinv, (((1,), (1,)), ((), ())),
                         precision=lax.Precision.HIGHEST,
                         preferred_element_type=jnp.float32)
    blks = [jnp.broadcast_to(dt[:, k:k + 1], (D, D))
            for k in range(R // D)]
    return jnp.concatenate(blks, axis=0)


def _matmul_body(x_ref, w_ref, o_ref):
    o_ref[...] = jnp.dot(x_ref[...], w_ref[...],
                         preferred_element_type=jnp.float32)


def _scale_body(deg_ref, h_ref, o_ref, o2_ref):
    v = h_ref[...] * _dinv_block(deg_ref)
    o_ref[...] = v
    o2_ref[...] = v


def _mid_body(deg_ref, agg_ref, hs_ref, w_ref, b_ref, o_ref, o2_ref):
    dinv = _dinv_block(deg_ref)
    t = (agg_ref[0] + agg_ref[1] + hs_ref[...]) * dinv + b_ref[...]
    v = jnp.dot(t, w_ref[...], preferred_element_type=jnp.float32) * dinv
    o_ref[...] = v
    o2_ref[...] = v


def _final_body(deg_ref, agg_ref, hs_ref, b_ref, o_ref):
    o_ref[...] = (agg_ref[0] + agg_ref[1] + hs_ref[...]) * _dinv_block(deg_ref) \
        + b_ref[...]


def kernel(x, edge_index, W1, b1, W2, b2):
    N = x.shape[0]
    E = edge_index.shape[1]
    n_pad = ((N + R - 1) // R) * R
    group = NC * NS * CH
    e_pad = ((E + group - 1) // group) * group
    pad = e_pad - E
    hrows = n_pad // D

    src = jnp.concatenate([edge_index[0], jnp.zeros((pad,), jnp.int32)])
    dst = jnp.concatenate([edge_index[1], jnp.full((pad,), N, jnp.int32)])
    x_pad = jnp.pad(x, ((0, n_pad - N), (0, 0)))
    zeros_agg = jnp.zeros((n_pad, D), jnp.float32)
    zeros_deg = jnp.zeros((hrows, D), jnp.float32)
    lin = jnp.arange(hrows, dtype=jnp.int32)
    w1t = W1.T
    w2t = W2.T
    b1r = b1.reshape(1, D)
    b2r = b2.reshape(1, D)

    deg_kernel = _make_deg_kernel(e_pad, n_pad)
    agg_kernel = _make_agg_kernel(e_pad, n_pad)
    grid = (n_pad // R,)

    deg2 = deg_kernel(dst, lin, zeros_deg).reshape(NC, hrows, D)

    h1 = pl.pallas_call(
        _matmul_body,
        grid=grid,
        in_specs=[pl.BlockSpec((R, D), lambda i: (i, 0)),
                  pl.BlockSpec((D, D), lambda i: (0, 0))],
        out_specs=pl.BlockSpec((R, D), lambda i: (i, 0)),
        out_shape=jax.ShapeDtypeStruct((n_pad, D), jnp.float32),
    )(x_pad, w1t)

    row_spec = pl.BlockSpec((R, D), lambda i: (i, 0))
    agg_spec = pl.BlockSpec((NC, R, D), lambda i: (0, i, 0))
    deg_spec = pl.BlockSpec((NC, R // D, D), lambda i: (0, i, 0))
    b_spec = pl.BlockSpec((1, D), lambda i: (0, 0))
    w_spec = pl.BlockSpec((D, D), lambda i: (0, 0))

    hs1, hs1b = pl.pallas_call(
        _scale_body,
        grid=grid,
        in_specs=[deg_spec, row_spec],
        out_specs=[row_spec, row_spec],
        out_shape=[jax.ShapeDtypeStruct((n_pad, D), jnp.float32)] * 2,
    )(deg2, h1)

    agg1 = agg_kernel(src, dst, hs1, hs1b, zeros_agg).reshape(NC, n_pad, D)

    hs2, hs2b = pl.pallas_call(
        _mid_body,
        grid=grid,
        in_specs=[deg_spec, agg_spec, row_spec, w_spec, b_spec],
        out_specs=[row_spec, row_spec],
        out_shape=[jax.ShapeDtypeStruct((n_pad, D), jnp.float32)] * 2,
    )(deg2, agg1, hs1, w2t, b1r)

    agg2 = agg_kernel(src, dst, hs2, hs2b, zeros_agg).reshape(NC, n_pad, D)

    out = pl.pallas_call(
        _final_body,
        grid=grid,
        in_specs=[deg_spec, agg_spec, row_spec, b_spec],
        out_specs=row_spec,
        out_shape=jax.ShapeDtypeStruct((n_pad, D), jnp.float32),
    )(deg2, agg2, hs2, b2r)

    return out[:N]


# speedup vs baseline: 1.6958x; 1.0727x over previous
import dataclasses
import functools

import jax
import jax.numpy as jnp
from jax import lax
from jax.experimental import pallas as pl
from jax.experimental.pallas import tpu as pltpu
from jax.experimental.pallas import tpu_sc as plsc

D = 128
CH = 128
NC = 2
NS = 16
L = 16
R = 1024


def _mesh():
    return plsc.VectorSubcoreMesh(core_axis_name="c", subcore_axis_name="s")


def _no_layout_params():
    cp = pltpu.CompilerParams()
    if "needs_layout_passes" in pltpu.CompilerParams.__dataclass_fields__:
        cp = dataclasses.replace(cp, needs_layout_passes=False)
    return cp


def _make_deg_kernel(e_pad, n_pad):
    edges_per_tile = e_pad // (NC * NS)
    chunks = edges_per_tile // CH
    hrows = n_pad // D
    wsub = hrows // 8

    @functools.partial(
        pl.kernel,
        out_type=jax.ShapeDtypeStruct((NC * hrows, D), jnp.float32),
        mesh=_mesh(),
        compiler_params=_no_layout_params(),
        scratch_types=[
            pltpu.VMEM((CH,), jnp.int32),
            pltpu.VMEM((hrows,), jnp.int32),
            pltpu.VMEM((hrows, D), jnp.float32),
            pltpu.VMEM_SHARED((hrows, D), jnp.float32),
            pltpu.SemaphoreType.DMA,
        ],
    )
    def deg_kernel(dst_hbm, lin_hbm, zeros_hbm, out_hbm,
                   didx, lin_v, hist, acc, sem):
        c = lax.axis_index("c")
        s = lax.axis_index("s")

        @pl.when(s == 0)
        def _():
            pltpu.sync_copy(zeros_hbm, acc)

        pltpu.sync_copy(lin_hbm, lin_v)
        pltpu.sync_copy(zeros_hbm, hist)
        plsc.subcore_barrier()

        base = (c * NS + s) * edges_per_tile
        ones16 = jnp.full((L,), 1.0, jnp.float32)

        @pl.loop(0, chunks)
        def _(j):
            pltpu.sync_copy(dst_hbm.at[pl.ds(base + j * CH, CH)], didx)
            for k in range(CH // L):
                v = didx[pl.ds(k * L, L)]
                row = lax.shift_right_logical(v, 7)
                col = lax.bitwise_and(v, 127)
                plsc.addupdate_scatter(hist, [row, col], ones16)

        pltpu.sync_copy(hist, acc.at[lin_v], add=True)
        plsc.subcore_barrier()

        @pl.when(s < wsub)
        def _():
            pltpu.sync_copy(
                acc.at[pl.ds(s * 8, 8)],
                out_hbm.at[pl.ds(c * hrows + s * 8, 8)],
            )

    return deg_kernel


def _make_agg_kernel(e_pad, n_pad):
    total_chunks = e_pad // CH
    chunks0 = (total_chunks * 58 // 100) // NS
    chunks1 = (total_chunks - chunks0 * NS) // NS
    assert (chunks0 + chunks1) * NS == total_chunks
    ept0 = chunks0 * CH
    ept1 = chunks1 * CH
    rows_per_sub = n_pad // NS

    @functools.partial(
        pl.kernel,
        out_type=jax.ShapeDtypeStruct((NC * n_pad, D), jnp.float32),
        mesh=_mesh(),
        scratch_types=[
            pltpu.VMEM((CH,), jnp.int32),
            pltpu.VMEM((CH,), jnp.int32),
            pltpu.VMEM((CH, D), jnp.float32),
            pltpu.VMEM_SHARED((n_pad, D), jnp.float32),
            pltpu.SemaphoreType.DMA,
        ],
    )
    def agg_kernel(src_hbm, dst_hbm, hs_hbm, zeros_hbm, out_hbm,
                   sidx, didx, rows, acc, sem):
        c = lax.axis_index("c")
        s = lax.axis_index("s")

        @pl.when(s == 0)
        def _():
            pltpu.sync_copy(zeros_hbm, acc)

        plsc.subcore_barrier()

        def run(base, nchunks):
            @pl.loop(0, nchunks)
            def _(j):
                off = base + j * CH
                pltpu.sync_copy(src_hbm.at[pl.ds(off, CH)], sidx)
                pltpu.sync_copy(dst_hbm.at[pl.ds(off, CH)], didx)
                pltpu.async_copy(hs_hbm.at[sidx], rows, sem).wait()
                pltpu.sync_copy(rows, acc.at[didx], add=True)

        @pl.when(c == 0)
        def _():
            run(s * ept0, chunks0)

        @pl.when(c == 1)
        def _():
            run(NS * ept0 + s * ept1, chunks1)

        plsc.subcore_barrier()
        pltpu.sync_copy(
            acc.at[pl.ds(s * rows_per_sub, rows_per_sub)],
            out_hbm.at[pl.ds(c * n_pad + s * rows_per_sub, rows_per_sub)],
        )

    return agg_kernel


def _dinv_block(deg_ref):
    d = deg_ref[0] + deg_ref[1] + 1.0
    dinv = lax.rsqrt(d)
    eye = (lax.broadcasted_iota(jnp.int32, (D, D), 0)
           == lax.broadcasted_iota(jnp.int32, (D, D), 1)).astype(jnp.float32)
    dt = lax.dot_general(eye, dinv, (((1,), (1,)), ((), ())),
                         precision=lax.Precision.HIGHEST,
                         preferred_element_type=jnp.float32)
    blks = [jnp.broadcast_to(dt[:, k:k + 1], (D, D))
            for k in range(R // D)]
    return jnp.concatenate(blks, axis=0)


def _matmul_body(x_ref, w_ref, o_ref):
    o_ref[...] = jnp.dot(x_ref[...], w_ref[...],
                         preferred_element_type=jnp.float32)


def _scale_body(deg_ref, h_ref, o_ref):
    o_ref[...] = h_ref[...] * _dinv_block(deg_ref)


def _mid_body(deg_ref, agg_ref, hs_ref, w_ref, b_ref, o_ref):
    dinv = _dinv_block(deg_ref)
    t = (agg_ref[0] + agg_ref[1] + hs_ref[...]) * dinv + b_ref[...]
    o_ref[...] = jnp.dot(t, w_ref[...],
                         preferred_element_type=jnp.float32) * dinv


def _final_body(deg_ref, agg_ref, hs_ref, b_ref, o_ref):
    o_ref[...] = (agg_ref[0] + agg_ref[1] + hs_ref[...]) * _dinv_block(deg_ref) \
        + b_ref[...]


def kernel(x, edge_index, W1, b1, W2, b2):
    N = x.shape[0]
    E = edge_index.shape[1]
    n_pad = ((N + R - 1) // R) * R
    group = NC * NS * CH
    e_pad = ((E + group - 1) // group) * group
    pad = e_pad - E
    hrows = n_pad // D

    src = jnp.concatenate([edge_index[0], jnp.zeros((pad,), jnp.int32)])
    dst = jnp.concatenate([edge_index[1], jnp.full((pad,), N, jnp.int32)])
    x_pad = jnp.pad(x, ((0, n_pad - N), (0, 0)))
    zeros_agg = jnp.zeros((n_pad, D), jnp.float32)
    zeros_deg = jnp.zeros((hrows, D), jnp.float32)
    lin = jnp.arange(hrows, dtype=jnp.int32)
    w1t = W1.T
    w2t = W2.T
    b1r = b1.reshape(1, D)
    b2r = b2.reshape(1, D)

    deg_kernel = _make_deg_kernel(e_pad, n_pad)
    agg_kernel = _make_agg_kernel(e_pad, n_pad)
    grid = (n_pad // R,)

    deg2 = deg_kernel(dst, lin, zeros_deg).reshape(NC, hrows, D)

    h1 = pl.pallas_call(
        _matmul_body,
        grid=grid,
        in_specs=[pl.BlockSpec((R, D), lambda i: (i, 0)),
                  pl.BlockSpec((D, D), lambda i: (0, 0))],
        out_specs=pl.BlockSpec((R, D), lambda i: (i, 0)),
        out_shape=jax.ShapeDtypeStruct((n_pad, D), jnp.float32),
    )(x_pad, w1t)

    row_spec = pl.BlockSpec((R, D), lambda i: (i, 0))
    agg_spec = pl.BlockSpec((NC, R, D), lambda i: (0, i, 0))
    deg_spec = pl.BlockSpec((NC, R // D, D), lambda i: (0, i, 0))
    b_spec = pl.BlockSpec((1, D), lambda i: (0, 0))
    w_spec = pl.BlockSpec((D, D), lambda i: (0, 0))

    hs1 = pl.pallas_call(
        _scale_body,
        grid=grid,
        in_specs=[deg_spec, row_spec],
        out_specs=row_spec,
        out_shape=jax.ShapeDtypeStruct((n_pad, D), jnp.float32),
    )(deg2, h1)

    agg1 = agg_kernel(src, dst, hs1, zeros_agg).reshape(NC, n_pad, D)

    hs2 = pl.pallas_call(
        _mid_body,
        grid=grid,
        in_specs=[deg_spec, agg_spec, row_spec, w_spec, b_spec],
        out_specs=row_spec,
        out_shape=jax.ShapeDtypeStruct((n_pad, D), jnp.float32),
    )(deg2, agg1, hs1, w2t, b1r)

    agg2 = agg_kernel(src, dst, hs2, zeros_agg).reshape(NC, n_pad, D)

    out = pl.pallas_call(
        _final_body,
        grid=grid,
        in_specs=[deg_spec, agg_spec, row_spec, b_spec],
        out_specs=row_spec,
        out_shape=jax.ShapeDtypeStruct((n_pad, D), jnp.float32),
    )(deg2, agg2, hs2, b2r)

    return out[:N]
